# Initial kernel scaffold; baseline (speedup 1.0000x reference)
import jax
import jax.numpy as jnp
from jax.experimental import pallas as pl


def kernel(feats, edge_index, edge_feat, etypes, Mw, Mb, Uw, Ub, gamma, beta, mix_w, mix_b, Lw, Lb, Wih, Whh, bih, bhh):
    raise NotImplementedError("write your pallas kernel here")


# speedup vs baseline: 2.1657x; 2.1657x over previous
import functools

import jax
import jax.numpy as jnp
from jax.experimental import pallas as pl
from jax.experimental.pallas import tpu as pltpu

N = 10000
E = 320000
D = 128
T = 4
TI = 32
TO = 32
DELTA = 1.0


def _bf16_round(a):
    u = jax.lax.bitcast_convert_type(a, jnp.uint32)
    r = u + jnp.uint32(0x7FFF) + ((u >> 16) & jnp.uint32(1))
    return jax.lax.bitcast_convert_type(r & jnp.uint32(0xFFFF0000), jnp.float32)


def kernel(feats, edge_index, edge_feat, etypes, Mw, Mb, Uw, Ub, gamma, beta,
           mix_w, mix_b, Lw, Lb, Wih, Whh, bih, bhh):
    src = edge_index[0]
    dst = edge_index[1]

    deg = jax.ops.segment_sum(jnp.ones((E,), jnp.float32), dst, num_segments=N)
    degc = jnp.maximum(deg, 1.0)[:, None]
    has = (deg > 0)[:, None]
    logd = jnp.log(deg + 1.0)[:, None]
    amp_f = logd / DELTA
    att_f = jnp.where(logd > 0, DELTA / jnp.where(logd > 0, logd, 1.0), 0.0)

    fb = _bf16_round(feats)
    HP = jax.lax.Precision.HIGHEST
    HA = jnp.concatenate(
        [jax.lax.dot(fb[:, t * TI:(t + 1) * TI], _bf16_round(Mw[t][:, :TI].T),
                     precision=HP) for t in range(T)], axis=-1)
    HB = jnp.concatenate(
        [jax.lax.dot(fb[:, t * TI:(t + 1) * TI], _bf16_round(Mw[t][:, TI:2 * TI].T),
                     precision=HP) for t in range(T)], axis=-1)
    cvec = _bf16_round(jnp.concatenate([Mw[t][:, 2 * TI] for t in range(T)], axis=0))
    bias = Mb.reshape(-1)

    efb = _bf16_round(edge_feat[:, 0:1])
    msg = HA[src] + HB[dst] + efb * cvec[None, :] + bias[None, :]
    s_f = jax.ops.segment_sum(msg, dst, num_segments=N)
    ssq_f = jax.ops.segment_sum(msg * msg, dst, num_segments=N)
    mx_f = jnp.where(has, jax.ops.segment_max(msg, dst, num_segments=N), 0.0)
    mn_f = jnp.where(has, -jax.ops.segment_max(-msg, dst, num_segments=N), 0.0)
    mean_f = s_f / degc
    var_f = jax.nn.relu(ssq_f / degc - mean_f * mean_f)
    std_f = jnp.sqrt(var_f + 1e-30)

    tower_outs = []
    for t in range(T):
        sl = slice(t * TI, (t + 1) * TI)
        h = feats[:, sl]
        agg = jnp.concatenate([s_f[:, sl], mean_f[:, sl], mx_f[:, sl],
                               mn_f[:, sl], std_f[:, sl]], axis=-1)
        h_neigh = jnp.concatenate([agg, agg * amp_f, agg * att_f], axis=-1)
        u_in = jnp.concatenate([h, h_neigh], axis=-1)
        out = u_in @ Uw[t].T + Ub[t]
        bm = jnp.mean(out, axis=0)
        bv = jnp.var(out, axis=0)
        out = gamma[t] * (out - bm) / jnp.sqrt(bv + 1e-5) + beta[t]
        tower_outs.append(out)
    h_cat = jnp.concatenate(tower_outs, axis=-1)
    emb1 = jax.nn.leaky_relu(h_cat @ mix_w.T + mix_b, negative_slope=0.01)

    h = emb1
    gidx = etypes * N + src
    for _ in range(2):
        H0 = h @ Lw[0].T + Lb[0]
        H1 = h @ Lw[1].T + Lb[1]
        HT = jnp.concatenate([H0, H1], axis=0)
        a = jax.ops.segment_sum(HT[gidx], dst, num_segments=N)
        gi = a @ Wih.T + bih
        gh = h @ Whh.T + bhh
        ir, iz, inn = jnp.split(gi, 3, axis=-1)
        hr, hz, hn = jnp.split(gh, 3, axis=-1)
        r = jax.nn.sigmoid(ir + hr)
        z = jax.nn.sigmoid(iz + hz)
        n = jnp.tanh(inn + r * hn)
        h = (1.0 - z) * n + z * h
    return (emb1, h)


# speedup vs baseline: 2.6448x; 1.2213x over previous
import functools

import jax
import jax.numpy as jnp
from jax import lax
from jax.experimental import pallas as pl
from jax.experimental.pallas import tpu as pltpu
from jax.experimental.pallas import tpu_sc as plsc

N = 10000
E = 320000
D = 128
T = 4
TI = 32
TO = 32
DELTA = 1.0

NPAD = 10240
E2 = 327680
KC = 128
RPT = NPAD // 16
EPW = E2 // 32


def _bf16_round(a):
    u = jax.lax.bitcast_convert_type(a, jnp.uint32)
    r = u + jnp.uint32(0x7FFF) + ((u >> 16) & jnp.uint32(1))
    return jax.lax.bitcast_convert_type(r & jnp.uint32(0xFFFF0000), jnp.float32)


def _seg_sum_sc():
    mesh = plsc.VectorSubcoreMesh(core_axis_name="c", subcore_axis_name="s")

    @functools.partial(
        pl.kernel, mesh=mesh,
        out_type=jax.ShapeDtypeStruct((2, NPAD, 128), jnp.float32),
        scratch_types=[
            pltpu.VMEM((KC,), jnp.int32),
            pltpu.VMEM((KC,), jnp.int32),
            pltpu.VMEM((KC, 128), jnp.float32),
            pltpu.VMEM_SHARED((NPAD, 128), jnp.float32),
            pltpu.SemaphoreType.DMA,
        ],
    )
    def k(tab_hbm, idx_hbm, dst_hbm, zeros_hbm, out_hbm, idx_v, dst_v, rows_v, acc, sem):
        c = lax.axis_index("c")
        s = lax.axis_index("s")
        rows0 = s * RPT
        pltpu.sync_copy(zeros_hbm.at[pl.ds(rows0, RPT)], acc.at[pl.ds(rows0, RPT)])
        plsc.subcore_barrier()
        ebase = (c * 16 + s) * EPW

        def body(i, carry):
            base = ebase + i * KC
            pltpu.sync_copy(idx_hbm.at[pl.ds(base, KC)], idx_v)
            pltpu.sync_copy(dst_hbm.at[pl.ds(base, KC)], dst_v)
            pltpu.async_copy(tab_hbm.at[idx_v], rows_v, sem).wait()
            pltpu.sync_copy(rows_v, acc.at[dst_v], add=True)
            return carry

        lax.fori_loop(0, EPW // KC, body, 0)
        plsc.subcore_barrier()
        pltpu.sync_copy(acc.at[pl.ds(rows0, RPT)], out_hbm.at[c, pl.ds(rows0, RPT)])

    return k


_SEG_SUM_SC = _seg_sum_sc()


def _ggc_segment_sum(HT, gpad, dst_p, zeros_pad):
    out = _SEG_SUM_SC(HT, gpad, dst_p, zeros_pad)
    return (out[0, :N, :] + out[1, :N, :])


def kernel(feats, edge_index, edge_feat, etypes, Mw, Mb, Uw, Ub, gamma, beta,
           mix_w, mix_b, Lw, Lb, Wih, Whh, bih, bhh):
    src = edge_index[0]
    dst = edge_index[1]

    deg = jax.ops.segment_sum(jnp.ones((E,), jnp.float32), dst, num_segments=N)
    degc = jnp.maximum(deg, 1.0)[:, None]
    has = (deg > 0)[:, None]
    logd = jnp.log(deg + 1.0)[:, None]
    amp_f = logd / DELTA
    att_f = jnp.where(logd > 0, DELTA / jnp.where(logd > 0, logd, 1.0), 0.0)

    fb = _bf16_round(feats)
    HP = jax.lax.Precision.HIGHEST
    HA = jnp.concatenate(
        [jax.lax.dot(fb[:, t * TI:(t + 1) * TI], _bf16_round(Mw[t][:, :TI].T),
                     precision=HP) for t in range(T)], axis=-1)
    HB = jnp.concatenate(
        [jax.lax.dot(fb[:, t * TI:(t + 1) * TI], _bf16_round(Mw[t][:, TI:2 * TI].T),
                     precision=HP) for t in range(T)], axis=-1)
    cvec = _bf16_round(jnp.concatenate([Mw[t][:, 2 * TI] for t in range(T)], axis=0))
    bias = Mb.reshape(-1)

    efb = _bf16_round(edge_feat[:, 0:1])
    msg = HA[src] + HB[dst] + efb * cvec[None, :] + bias[None, :]
    s_f = jax.ops.segment_sum(msg, dst, num_segments=N)
    ssq_f = jax.ops.segment_sum(msg * msg, dst, num_segments=N)
    mx_f = jnp.where(has, jax.ops.segment_max(msg, dst, num_segments=N), 0.0)
    mn_f = jnp.where(has, -jax.ops.segment_max(-msg, dst, num_segments=N), 0.0)
    mean_f = s_f / degc
    var_f = jax.nn.relu(ssq_f / degc - mean_f * mean_f)
    std_f = jnp.sqrt(var_f + 1e-30)

    tower_outs = []
    for t in range(T):
        sl = slice(t * TI, (t + 1) * TI)
        h = feats[:, sl]
        agg = jnp.concatenate([s_f[:, sl], mean_f[:, sl], mx_f[:, sl],
                               mn_f[:, sl], std_f[:, sl]], axis=-1)
        h_neigh = jnp.concatenate([agg, agg * amp_f, agg * att_f], axis=-1)
        u_in = jnp.concatenate([h, h_neigh], axis=-1)
        out = u_in @ Uw[t].T + Ub[t]
        bm = jnp.mean(out, axis=0)
        bv = jnp.var(out, axis=0)
        out = gamma[t] * (out - bm) / jnp.sqrt(bv + 1e-5) + beta[t]
        tower_outs.append(out)
    h_cat = jnp.concatenate(tower_outs, axis=-1)
    emb1 = jax.nn.leaky_relu(h_cat @ mix_w.T + mix_b, negative_slope=0.01)

    h = emb1
    gidx = etypes * N + src
    gpad = jnp.pad(gidx, (0, E2 - E))
    dst_p = jnp.pad(dst, (0, E2 - E), constant_values=NPAD - 1)
    zeros_pad = jnp.zeros((NPAD, 128), jnp.float32)
    for _ in range(2):
        H0 = h @ Lw[0].T + Lb[0]
        H1 = h @ Lw[1].T + Lb[1]
        HT = jnp.concatenate([H0, H1], axis=0)
        a = _ggc_segment_sum(HT, gpad, dst_p, zeros_pad)
        gi = a @ Wih.T + bih
        gh = h @ Whh.T + bhh
        ir, iz, inn = jnp.split(gi, 3, axis=-1)
        hr, hz, hn = jnp.split(gh, 3, axis=-1)
        r = jax.nn.sigmoid(ir + hr)
        z = jax.nn.sigmoid(iz + hz)
        n = jnp.tanh(inn + r * hn)
        h = (1.0 - z) * n + z * h
    return (emb1, h)


# speedup vs baseline: 2.6733x; 1.0108x over previous
import functools

import jax
import jax.numpy as jnp
from jax import lax
from jax.experimental import pallas as pl
from jax.experimental.pallas import tpu as pltpu
from jax.experimental.pallas import tpu_sc as plsc

N = 10000
E = 320000
D = 128
T = 4
TI = 32
TO = 32
DELTA = 1.0

NPAD = 10240
E2 = 327680
KC = 128
RPT = NPAD // 16
EPW = E2 // 32


def _bf16_round(a):
    u = jax.lax.bitcast_convert_type(a, jnp.uint32)
    r = u + jnp.uint32(0x7FFF) + ((u >> 16) & jnp.uint32(1))
    return jax.lax.bitcast_convert_type(r & jnp.uint32(0xFFFF0000), jnp.float32)


def _seg_sum_sc():
    mesh = plsc.VectorSubcoreMesh(core_axis_name="c", subcore_axis_name="s")

    @functools.partial(
        pl.kernel, mesh=mesh,
        out_type=jax.ShapeDtypeStruct((2, NPAD, 128), jnp.float32),
        scratch_types=[
            pltpu.VMEM((KC,), jnp.int32),
            pltpu.VMEM((KC,), jnp.int32),
            pltpu.VMEM((KC, 128), jnp.float32),
            pltpu.VMEM_SHARED((NPAD, 128), jnp.float32),
            pltpu.SemaphoreType.DMA,
        ],
    )
    def k(tab_hbm, idx_hbm, dst_hbm, zeros_hbm, out_hbm, idx_v, dst_v, rows_v, acc, sem):
        c = lax.axis_index("c")
        s = lax.axis_index("s")
        rows0 = s * RPT
        pltpu.sync_copy(zeros_hbm.at[pl.ds(rows0, RPT)], acc.at[pl.ds(rows0, RPT)])
        plsc.subcore_barrier()
        ebase = (c * 16 + s) * EPW

        def body(i, carry):
            base = ebase + i * KC
            pltpu.sync_copy(idx_hbm.at[pl.ds(base, KC)], idx_v)
            pltpu.sync_copy(dst_hbm.at[pl.ds(base, KC)], dst_v)
            pltpu.async_copy(tab_hbm.at[idx_v], rows_v, sem).wait()
            pltpu.sync_copy(rows_v, acc.at[dst_v], add=True)
            return carry

        lax.fori_loop(0, EPW // KC, body, 0)
        plsc.subcore_barrier()
        pltpu.sync_copy(acc.at[pl.ds(rows0, RPT)], out_hbm.at[c, pl.ds(rows0, RPT)])

    return k


_SEG_SUM_SC = _seg_sum_sc()


def _ggc_segment_sum(HT, gpad, dst_p, zeros_pad):
    out = _SEG_SUM_SC(HT, gpad, dst_p, zeros_pad)
    return (out[0, :N, :] + out[1, :N, :])


def kernel(feats, edge_index, edge_feat, etypes, Mw, Mb, Uw, Ub, gamma, beta,
           mix_w, mix_b, Lw, Lb, Wih, Whh, bih, bhh):
    src = edge_index[0]
    dst = edge_index[1]

    deg = jax.ops.segment_sum(jnp.ones((E,), jnp.float32), dst, num_segments=N)
    degc = jnp.maximum(deg, 1.0)[:, None]
    has = (deg > 0)[:, None]
    logd = jnp.log(deg + 1.0)[:, None]
    amp_f = logd / DELTA
    att_f = jnp.where(logd > 0, DELTA / jnp.where(logd > 0, logd, 1.0), 0.0)

    fb = _bf16_round(feats)
    HP = jax.lax.Precision.HIGHEST
    HA = jnp.concatenate(
        [jax.lax.dot(fb[:, t * TI:(t + 1) * TI], _bf16_round(Mw[t][:, :TI].T),
                     precision=HP) for t in range(T)], axis=-1)
    HB = jnp.concatenate(
        [jax.lax.dot(fb[:, t * TI:(t + 1) * TI], _bf16_round(Mw[t][:, TI:2 * TI].T),
                     precision=HP) for t in range(T)], axis=-1)
    cvec = _bf16_round(jnp.concatenate([Mw[t][:, 2 * TI] for t in range(T)], axis=0))
    bias = Mb.reshape(-1)

    efb = _bf16_round(edge_feat[:, 0:1])
    msg = HA[src] + HB[dst] + efb * cvec[None, :] + bias[None, :]
    eye_p = jnp.pad(jnp.arange(E, dtype=jnp.int32), (0, E2 - E))
    dst_p = jnp.pad(dst, (0, E2 - E), constant_values=NPAD - 1)
    zeros_pad = jnp.zeros((NPAD, 128), jnp.float32)
    s_out = _SEG_SUM_SC(msg, eye_p, dst_p, zeros_pad)
    s_f = s_out[0, :N, :] + s_out[1, :N, :]
    q_out = _SEG_SUM_SC(msg * msg, eye_p, dst_p, zeros_pad)
    ssq_f = q_out[0, :N, :] + q_out[1, :N, :]
    mx_f = jnp.where(has, jax.ops.segment_max(msg, dst, num_segments=N), 0.0)
    mn_f = jnp.where(has, -jax.ops.segment_max(-msg, dst, num_segments=N), 0.0)
    mean_f = s_f / degc
    var_f = jax.nn.relu(ssq_f / degc - mean_f * mean_f)
    std_f = jnp.sqrt(var_f + 1e-30)

    tower_outs = []
    for t in range(T):
        sl = slice(t * TI, (t + 1) * TI)
        h = feats[:, sl]
        agg = jnp.concatenate([s_f[:, sl], mean_f[:, sl], mx_f[:, sl],
                               mn_f[:, sl], std_f[:, sl]], axis=-1)
        h_neigh = jnp.concatenate([agg, agg * amp_f, agg * att_f], axis=-1)
        u_in = jnp.concatenate([h, h_neigh], axis=-1)
        out = u_in @ Uw[t].T + Ub[t]
        bm = jnp.mean(out, axis=0)
        bv = jnp.var(out, axis=0)
        out = gamma[t] * (out - bm) / jnp.sqrt(bv + 1e-5) + beta[t]
        tower_outs.append(out)
    h_cat = jnp.concatenate(tower_outs, axis=-1)
    emb1 = jax.nn.leaky_relu(h_cat @ mix_w.T + mix_b, negative_slope=0.01)

    h = emb1
    gidx = etypes * N + src
    gpad = jnp.pad(gidx, (0, E2 - E))
    for _ in range(2):
        H0 = h @ Lw[0].T + Lb[0]
        H1 = h @ Lw[1].T + Lb[1]
        HT = jnp.concatenate([H0, H1], axis=0)
        a = _ggc_segment_sum(HT, gpad, dst_p, zeros_pad)
        gi = a @ Wih.T + bih
        gh = h @ Whh.T + bhh
        ir, iz, inn = jnp.split(gi, 3, axis=-1)
        hr, hz, hn = jnp.split(gh, 3, axis=-1)
        r = jax.nn.sigmoid(ir + hr)
        z = jax.nn.sigmoid(iz + hz)
        n = jnp.tanh(inn + r * hn)
        h = (1.0 - z) * n + z * h
    return (emb1, h)
